# baseline (speedup 1.0000x reference)
import functools

import jax
import jax.numpy as jnp
from jax import lax
from jax.experimental import pallas as pl
from jax.experimental.pallas import tpu as pltpu
from jax.experimental.pallas import tpu_sc as plsc

INV_SQRT2 = 0.7071067811865476

NC = 2
NS = 16
NW = NC * NS


def _mesh():
    return plsc.VectorSubcoreMesh(core_axis_name="c", subcore_axis_name="s",
                                  num_cores=NC, num_subcores=NS)


def _pack8(g):
    n = g.shape[0] // 8
    a = g.reshape(n, 8, 16)
    return jnp.concatenate([a[:, k, :] for k in range(8)], axis=1)


def _unpack8(p):
    n = p.shape[0]
    parts = [p[:, 16 * k:16 * (k + 1)].reshape(n, 1, 16) for k in range(8)]
    return jnp.concatenate(parts, axis=1).reshape(n * 8, 16)


def _down_body(m_ref, rbf_ref, wm_ref, wr_ref, wdbd_ref, out_ref):
    h = jax.nn.silu(jnp.dot(m_ref[...], wm_ref[...],
                            preferred_element_type=jnp.float32))
    Rr = out_ref.shape[0]
    h3 = h.reshape(Rr, 8, 128)
    hw = jnp.concatenate([h3[:, k, :] for k in range(8)], axis=1)
    rp = jnp.dot(rbf_ref[...], wr_ref[...], preferred_element_type=jnp.float32)
    rp3 = rp.reshape(Rr, 8, 128)
    rpw = jnp.concatenate([rp3[:, k, :] for k in range(8)], axis=1)
    z = (hw * rpw).astype(jnp.bfloat16)
    out_ref[...] = jax.nn.silu(jnp.dot(z, wdbd_ref[...],
                                       preferred_element_type=jnp.float32))


def _stage_down(m_st, rbf, W_mkt, W_rbf, Wd_bd):
    E, DE = m_st.shape
    R = 3200
    return pl.pallas_call(
        _down_body,
        grid=(E // R,),
        in_specs=[
            pl.BlockSpec((R, DE), lambda i: (i, 0)),
            pl.BlockSpec((R, 16), lambda i: (i, 0)),
            pl.BlockSpec(W_mkt.shape, lambda i: (0, 0)),
            pl.BlockSpec(W_rbf.shape, lambda i: (0, 0)),
            pl.BlockSpec(Wd_bd.shape, lambda i: (0, 0)),
        ],
        out_specs=pl.BlockSpec((R // 8, 128), lambda i: (i, 0)),
        out_shape=jax.ShapeDtypeStruct((E // 8, 128), jnp.float32),
    )(m_st, rbf, W_mkt, W_rbf, Wd_bd)


def _bil_body(cbf_ref, mt_ref, ids_ref, w2bd_ref, krbd_ref, ssbd_ref,
              y_ref, rem_ref):
    cbfp = _pack8(cbf_ref[...])
    P = jnp.dot(cbfp.astype(jnp.bfloat16), krbd_ref[...],
                preferred_element_type=jnp.float32)
    M = jnp.dot(mt_ref[...].astype(jnp.bfloat16), w2bd_ref[...],
                preferred_element_type=jnp.float32)
    Z = (P * M).astype(jnp.bfloat16)
    y_ref[...] = jnp.dot(Z, ssbd_ref[...], preferred_element_type=jnp.float32)
    ids = ids_ref[...]
    for r in range(4):
        lo = r * _ER
        inr = (ids >= lo) & (ids < lo + _ER)
        rem_ref[r, :, :] = jnp.where(inr, ids - lo, _DUMP)


_IDL = 128


def _stage_bilinear(cbf, m_t_packed, ids2d, W2_bd, Kr_bd, Ss_bd):
    T = cbf.shape[0]
    R = 5120
    return pl.pallas_call(
        _bil_body,
        grid=(T // R,),
        in_specs=[
            pl.BlockSpec((R, 16), lambda i: (i, 0)),
            pl.BlockSpec((R // 8, 128), lambda i: (i, 0)),
            pl.BlockSpec((R // _IDL, _IDL), lambda i: (i, 0)),
            pl.BlockSpec(W2_bd.shape, lambda i: (0, 0)),
            pl.BlockSpec(Kr_bd.shape, lambda i: (0, 0)),
            pl.BlockSpec(Ss_bd.shape, lambda i: (0, 0)),
        ],
        out_specs=[
            pl.BlockSpec((R // 8, 128), lambda i: (i, 0)),
            pl.BlockSpec((4, R // _IDL, _IDL), lambda i: (0, i, 0)),
        ],
        out_shape=[
            jax.ShapeDtypeStruct((T // 8, 128), jnp.float32),
            jax.ShapeDtypeStruct((4, T // _IDL, _IDL), jnp.int32),
        ],
    )(cbf, m_t_packed, ids2d, W2_bd, Kr_bd, Ss_bd)


def _out_body(x_ref, xsw_ref, wst_ref, wts_ref, out_ref):
    xp = x_ref[...]
    xsp = xsw_ref[...]
    wst = wst_ref[...]
    wts = wts_ref[...]
    Rr = xp.shape[0]
    outs = []
    for k in range(8):
        a = jax.nn.silu(jnp.dot(xp[:, 16 * k:16 * (k + 1)], wst,
                                preferred_element_type=jnp.float32))
        b = jax.nn.silu(jnp.dot(xsp[:, 16 * k:16 * (k + 1)], wts,
                                preferred_element_type=jnp.float32))
        outs.append(((a + b) * INV_SQRT2).reshape(Rr, 1, wst.shape[1]))
    out_ref[...] = jnp.concatenate(outs, axis=1).reshape(out_ref.shape)


def _stage_out(x_packed, x_sw_packed, W_st, W_ts):
    E = x_packed.shape[0] * 8
    DE = W_st.shape[1]
    R = 3200
    return pl.pallas_call(
        _out_body,
        grid=(E // R,),
        in_specs=[
            pl.BlockSpec((R // 8, 128), lambda i: (i, 0)),
            pl.BlockSpec((R // 8, 128), lambda i: (i, 0)),
            pl.BlockSpec(W_st.shape, lambda i: (0, 0)),
            pl.BlockSpec(W_ts.shape, lambda i: (0, 0)),
        ],
        out_specs=pl.BlockSpec((R, DE), lambda i: (i, 0)),
        out_shape=jax.ShapeDtypeStruct((E, DE), jnp.float32),
    )(x_packed, x_sw_packed, W_st, W_ts)


def _gather_rows(table, idx, chunk, k):
    n = idx.shape[0]
    d = table.shape[1]
    per_w = n // NW
    n_groups = per_w // (chunk * k)

    @functools.partial(
        pl.kernel,
        out_type=jax.ShapeDtypeStruct((n, d), jnp.float32),
        mesh=_mesh(),
        compiler_params=pltpu.CompilerParams(use_tc_tiling_on_sc=False),
        scratch_types=[
            pltpu.VMEM((per_w,), jnp.int32),
            pltpu.VMEM((k * chunk, d), jnp.float32),
            pltpu.SemaphoreType.DMA,
        ],
    )
    def gk(table_hbm, idx_hbm, out_hbm, idx_v, buf, sem):
        wid = lax.axis_index("s") * NC + lax.axis_index("c")
        base = pl.multiple_of(wid * per_w, 8)
        pltpu.sync_copy(idx_hbm.at[pl.ds(base, per_w)], idx_v)

        def group(g, carry):
            descs = []
            for b in range(k):
                off = pl.multiple_of(g * (k * chunk) + b * chunk, 8)
                descs.append(pltpu.async_copy(
                    table_hbm.at[idx_v.at[pl.ds(off, chunk)]],
                    buf.at[pl.ds(b * chunk, chunk)], sem))
            for dsc in descs:
                dsc.wait()
            o = pl.multiple_of(base + g * (k * chunk), 8)
            pltpu.sync_copy(buf, out_hbm.at[pl.ds(o, k * chunk)])
            return carry

        lax.fori_loop(0, n_groups, group, 0)

    return gk(table, idx)


_ER = 80000
_ACC = 81920
_DUMP = 81000
_SCH = 800
_SJ = _SCH // 80


def _scatter_sum(y, rem, E):
    T, d = y.shape
    per_tile = T // NS
    half = per_tile // 2
    n_pass = E // (_ER * NC)

    @functools.partial(
        pl.kernel,
        out_type=jax.ShapeDtypeStruct((E, d), jnp.float32),
        mesh=_mesh(),
        compiler_params=pltpu.CompilerParams(use_tc_tiling_on_sc=False,
                                             needs_layout_passes=False),
        scratch_types=[
            pltpu.VMEM((half,), jnp.int32),
            pltpu.VMEM((1600,), jnp.int32),
            pltpu.VMEM((1600,), jnp.int32),
            pltpu.VMEM((800, d), jnp.float32),
            pltpu.VMEM((512, d), jnp.float32),
            pltpu.VMEM_SHARED((_ACC, d), jnp.float32),
            pltpu.SemaphoreType.DMA,
            pltpu.SemaphoreType.DMA,
        ],
    )
    def sk(y_hbm, rem_hbm, out_hbm, ib, locl, tixl, stg, zbuf, acc,
           lsem, ssem):
        c = lax.axis_index("c")
        s = lax.axis_index("s")
        t0 = s * per_tile
        iota = lax.iota(jnp.int32, 16)
        zero16 = jnp.zeros((d,), jnp.float32)

        def zrow(i, carry):
            zbuf[i, :] = zero16
            return carry
        lax.fori_loop(0, 512, zrow, 0)

        def fire():
            pltpu.async_copy(y_hbm.at[tixl.at[pl.ds(0, 800)]], stg,
                             lsem).wait()
            pltpu.async_copy(stg, acc.at[locl.at[pl.ds(0, 800)]], ssem,
                             add=True).wait()
            for j in range(50):
                o = 800 + j * 16
                locl[pl.ds(j * 16, 16)] = locl[pl.ds(o, 16)]
                tixl[pl.ds(j * 16, 16)] = tixl[pl.ds(o, 16)]

        for p in range(n_pass):
            rng = p * NC + c
            for zi in range(10):
                z0 = pl.multiple_of(s * 5120 + zi * 512, 8)
                pltpu.sync_copy(zbuf, acc.at[pl.ds(z0, 512)])
            plsc.subcore_barrier()

            pend = jnp.zeros((16,), jnp.int32)
            for hh in range(2):
                toff = pl.multiple_of(t0 + hh * half, 8)
                pltpu.sync_copy(rem_hbm.at[rng, pl.ds(toff, half)], ib)

                def step(j, pending):
                    v = ib[pl.ds(j * 16, 16)]
                    m = v < _ER
                    mi = m.astype(jnp.int32)
                    _, sv = plsc.sort_key_val(mi, v, descending=True)
                    tix = toff + j * 16 + iota
                    _, stx = plsc.sort_key_val(mi, tix, descending=True)
                    wpos = pending + iota
                    plsc.store_scatter(locl, [wpos], sv)
                    plsc.store_scatter(tixl, [wpos], stx)
                    pending = pending + plsc.all_reduce_population_count(m)
                    full = pending >= 800

                    @pl.when(jnp.any(full))
                    def _():
                        fire()
                    return jnp.where(full, pending - 800, pending)
                pend = lax.fori_loop(0, half // 16, step, pend)

            @pl.when(jnp.any(pend > 0))
            def _():
                dumpv = jnp.full((16,), _DUMP, jnp.int32)
                zv = jnp.zeros((16,), jnp.int32)
                for j in range(50):
                    padpos = pend + j * 16 + iota
                    plsc.store_scatter(locl, [padpos], dumpv)
                    plsc.store_scatter(tixl, [padpos], zv)
                fire()
            plsc.subcore_barrier()

            o0 = pl.multiple_of(s * 5000, 8)
            pltpu.sync_copy(acc.at[pl.ds(o0, 5000)],
                            out_hbm.at[pl.ds(rng * _ER + o0, 5000)])
            plsc.subcore_barrier()

    return sk(y, rem)


def kernel(m_st, rbf, cbf, idx_swap, id3_kt, id3_st, id3_ragged_idx,
           W_mkt, W_rbf, W_down, W_bil, W_st, W_ts):
    del id3_ragged_idx
    E = m_st.shape[0]
    T = cbf.shape[0]
    eye8 = jnp.eye(8, dtype=jnp.float32)
    Wd_bd = jnp.kron(eye8, W_down).astype(jnp.bfloat16)
    m16 = _stage_down(m_st, rbf, W_mkt, W_rbf, Wd_bd).reshape(E, 16)
    m_t = _gather_rows(m16, id3_kt, chunk=80, k=10)
    W2 = W_bil.reshape(16, 256)
    eye = jnp.eye(16, dtype=jnp.float32)
    Krep = jnp.repeat(eye, 16, axis=1)
    Ssum = jnp.tile(eye, (16, 1))
    W2_bd = jnp.kron(eye8, W2).astype(jnp.bfloat16)
    Kr_bd = jnp.kron(eye8, Krep).astype(jnp.bfloat16)
    Ss_bd = jnp.kron(eye8, Ssum).astype(jnp.bfloat16)
    ids2d = id3_st.reshape(T // _IDL, _IDL)
    y, rem = _stage_bilinear(cbf, m_t.reshape(T // 8, 128), ids2d,
                             W2_bd, Kr_bd, Ss_bd)
    x = _scatter_sum(y.reshape(T, 16), rem.reshape(4, T), E)
    x_sw = _gather_rows(x, idx_swap, chunk=80, k=5)
    return _stage_out(x.reshape(E // 8, 128), x_sw.reshape(E // 8, 128),
                      W_st, W_ts)


# speedup vs baseline: 1.0188x; 1.0188x over previous
import functools

import jax
import jax.numpy as jnp
from jax import lax
from jax.experimental import pallas as pl
from jax.experimental.pallas import tpu as pltpu
from jax.experimental.pallas import tpu_sc as plsc

INV_SQRT2 = 0.7071067811865476

NC = 2
NS = 16
NW = NC * NS


def _mesh():
    return plsc.VectorSubcoreMesh(core_axis_name="c", subcore_axis_name="s",
                                  num_cores=NC, num_subcores=NS)


def _pack8(g):
    n = g.shape[0] // 8
    a = g.reshape(n, 8, 16)
    return jnp.concatenate([a[:, k, :] for k in range(8)], axis=1)


def _unpack8(p):
    n = p.shape[0]
    parts = [p[:, 16 * k:16 * (k + 1)].reshape(n, 1, 16) for k in range(8)]
    return jnp.concatenate(parts, axis=1).reshape(n * 8, 16)


def _down_body(m_ref, rbfp_ref, wm_ref, wrbd_ref, wdbd_ref, out_ref):
    h = jax.nn.silu(jnp.dot(m_ref[...], wm_ref[...],
                            preferred_element_type=jnp.float32))
    Rr = out_ref.shape[0]
    h3 = h.reshape(Rr, 8, 128)
    hw = jnp.concatenate([h3[:, k, :] for k in range(8)], axis=1)
    rall = jnp.dot(rbfp_ref[...].astype(jnp.bfloat16), wrbd_ref[...],
                   preferred_element_type=jnp.float32)
    z = (hw * rall).astype(jnp.bfloat16)
    out_ref[...] = jax.nn.silu(jnp.dot(z, wdbd_ref[...],
                                       preferred_element_type=jnp.float32))


def _stage_down(m_st, rbf_p, W_mkt, Wr_bd, Wd_bd):
    E, DE = m_st.shape
    R = 3200
    return pl.pallas_call(
        _down_body,
        grid=(E // R,),
        in_specs=[
            pl.BlockSpec((R, DE), lambda i: (i, 0)),
            pl.BlockSpec((R // 8, 128), lambda i: (i, 0)),
            pl.BlockSpec(W_mkt.shape, lambda i: (0, 0)),
            pl.BlockSpec(Wr_bd.shape, lambda i: (0, 0)),
            pl.BlockSpec(Wd_bd.shape, lambda i: (0, 0)),
        ],
        out_specs=pl.BlockSpec((R // 8, 128), lambda i: (i, 0)),
        out_shape=jax.ShapeDtypeStruct((E // 8, 128), jnp.float32),
    )(m_st, rbf_p, W_mkt, Wr_bd, Wd_bd)


def _bil_body(cbf_ref, mt_ref, ids_ref, w2bd_ref, krbd_ref, ssbd_ref,
              y_ref, rem_ref):
    cbfp = _pack8(cbf_ref[...])
    P = jnp.dot(cbfp.astype(jnp.bfloat16), krbd_ref[...],
                preferred_element_type=jnp.float32)
    M = jnp.dot(mt_ref[...].astype(jnp.bfloat16), w2bd_ref[...],
                preferred_element_type=jnp.float32)
    Z = (P * M).astype(jnp.bfloat16)
    y_ref[...] = jnp.dot(Z, ssbd_ref[...], preferred_element_type=jnp.float32)
    ids = ids_ref[...]
    for r in range(4):
        lo = r * _ER
        inr = (ids >= lo) & (ids < lo + _ER)
        rem_ref[r, :, :] = jnp.where(inr, ids - lo, _DUMP)


_IDL = 128


def _stage_bilinear(cbf, m_t_packed, ids2d, W2_bd, Kr_bd, Ss_bd):
    T = cbf.shape[0]
    R = 5120
    return pl.pallas_call(
        _bil_body,
        grid=(T // R,),
        in_specs=[
            pl.BlockSpec((R, 16), lambda i: (i, 0)),
            pl.BlockSpec((R // 8, 128), lambda i: (i, 0)),
            pl.BlockSpec((R // _IDL, _IDL), lambda i: (i, 0)),
            pl.BlockSpec(W2_bd.shape, lambda i: (0, 0)),
            pl.BlockSpec(Kr_bd.shape, lambda i: (0, 0)),
            pl.BlockSpec(Ss_bd.shape, lambda i: (0, 0)),
        ],
        out_specs=[
            pl.BlockSpec((R // 8, 128), lambda i: (i, 0)),
            pl.BlockSpec((4, R // _IDL, _IDL), lambda i: (0, i, 0)),
        ],
        out_shape=[
            jax.ShapeDtypeStruct((T // 8, 128), jnp.float32),
            jax.ShapeDtypeStruct((4, T // _IDL, _IDL), jnp.int32),
        ],
    )(cbf, m_t_packed, ids2d, W2_bd, Kr_bd, Ss_bd)


def _out_body(x_ref, xsw_ref, wst_ref, wts_ref, out_ref):
    xp = x_ref[...]
    xsp = xsw_ref[...]
    wst = wst_ref[...]
    wts = wts_ref[...]
    Rr = xp.shape[0]
    outs = []
    for k in range(8):
        a = jax.nn.silu(jnp.dot(xp[:, 16 * k:16 * (k + 1)], wst,
                                preferred_element_type=jnp.float32))
        b = jax.nn.silu(jnp.dot(xsp[:, 16 * k:16 * (k + 1)], wts,
                                preferred_element_type=jnp.float32))
        outs.append(((a + b) * INV_SQRT2).reshape(Rr, 1, wst.shape[1]))
    out_ref[...] = jnp.concatenate(outs, axis=1).reshape(out_ref.shape)


def _stage_out(x_packed, x_sw_packed, W_st, W_ts):
    E = x_packed.shape[0] * 8
    DE = W_st.shape[1]
    R = 3200
    return pl.pallas_call(
        _out_body,
        grid=(E // R,),
        in_specs=[
            pl.BlockSpec((R // 8, 128), lambda i: (i, 0)),
            pl.BlockSpec((R // 8, 128), lambda i: (i, 0)),
            pl.BlockSpec(W_st.shape, lambda i: (0, 0)),
            pl.BlockSpec(W_ts.shape, lambda i: (0, 0)),
        ],
        out_specs=pl.BlockSpec((R, DE), lambda i: (i, 0)),
        out_shape=jax.ShapeDtypeStruct((E, DE), jnp.float32),
    )(x_packed, x_sw_packed, W_st, W_ts)


def _gather_rows(table, idx, chunk, k):
    n = idx.shape[0]
    d = table.shape[1]
    per_w = n // NW
    n_groups = per_w // (chunk * k)

    @functools.partial(
        pl.kernel,
        out_type=jax.ShapeDtypeStruct((n, d), jnp.float32),
        mesh=_mesh(),
        compiler_params=pltpu.CompilerParams(use_tc_tiling_on_sc=False),
        scratch_types=[
            pltpu.VMEM((per_w,), jnp.int32),
            pltpu.VMEM((k * chunk, d), jnp.float32),
            pltpu.SemaphoreType.DMA,
        ],
    )
    def gk(table_hbm, idx_hbm, out_hbm, idx_v, buf, sem):
        wid = lax.axis_index("s") * NC + lax.axis_index("c")
        base = pl.multiple_of(wid * per_w, 8)
        pltpu.sync_copy(idx_hbm.at[pl.ds(base, per_w)], idx_v)

        def group(g, carry):
            descs = []
            for b in range(k):
                off = pl.multiple_of(g * (k * chunk) + b * chunk, 8)
                descs.append(pltpu.async_copy(
                    table_hbm.at[idx_v.at[pl.ds(off, chunk)]],
                    buf.at[pl.ds(b * chunk, chunk)], sem))
            for dsc in descs:
                dsc.wait()
            o = pl.multiple_of(base + g * (k * chunk), 8)
            pltpu.sync_copy(buf, out_hbm.at[pl.ds(o, k * chunk)])
            return carry

        lax.fori_loop(0, n_groups, group, 0)

    return gk(table, idx)


_ER = 80000
_ACC = 81920
_DUMP = 81000
_SCH = 800
_SJ = _SCH // 80


def _scatter_sum(y, rem, E):
    T, d = y.shape
    per_tile = T // NS
    half = per_tile // 2
    n_pass = E // (_ER * NC)

    @functools.partial(
        pl.kernel,
        out_type=jax.ShapeDtypeStruct((E, d), jnp.float32),
        mesh=_mesh(),
        compiler_params=pltpu.CompilerParams(use_tc_tiling_on_sc=False,
                                             needs_layout_passes=False),
        scratch_types=[
            pltpu.VMEM((half,), jnp.int32),
            pltpu.VMEM((1600,), jnp.int32),
            pltpu.VMEM((1600,), jnp.int32),
            pltpu.VMEM((800, d), jnp.float32),
            pltpu.VMEM((512, d), jnp.float32),
            pltpu.VMEM_SHARED((_ACC, d), jnp.float32),
            pltpu.SemaphoreType.DMA,
            pltpu.SemaphoreType.DMA,
        ],
    )
    def sk(y_hbm, rem_hbm, out_hbm, ib, locl, tixl, stg, zbuf, acc,
           lsem, ssem):
        c = lax.axis_index("c")
        s = lax.axis_index("s")
        t0 = s * per_tile
        iota = lax.iota(jnp.int32, 16)
        zero16 = jnp.zeros((d,), jnp.float32)

        def zrow(i, carry):
            zbuf[i, :] = zero16
            return carry
        lax.fori_loop(0, 512, zrow, 0)

        def fire():
            pltpu.async_copy(y_hbm.at[tixl.at[pl.ds(0, 800)]], stg,
                             lsem).wait()
            pltpu.async_copy(stg, acc.at[locl.at[pl.ds(0, 800)]], ssem,
                             add=True).wait()
            for j in range(50):
                o = 800 + j * 16
                locl[pl.ds(j * 16, 16)] = locl[pl.ds(o, 16)]
                tixl[pl.ds(j * 16, 16)] = tixl[pl.ds(o, 16)]

        for p in range(n_pass):
            rng = p * NC + c
            for zi in range(10):
                z0 = pl.multiple_of(s * 5120 + zi * 512, 8)
                pltpu.sync_copy(zbuf, acc.at[pl.ds(z0, 512)])
            plsc.subcore_barrier()

            pend = jnp.zeros((16,), jnp.int32)
            for hh in range(2):
                toff = pl.multiple_of(t0 + hh * half, 8)
                pltpu.sync_copy(rem_hbm.at[rng, pl.ds(toff, half)], ib)

                def step(j, pending):
                    v = ib[pl.ds(j * 16, 16)]
                    m = v < _ER
                    mi = m.astype(jnp.int32)
                    _, sv = plsc.sort_key_val(mi, v, descending=True)
                    tix = toff + j * 16 + iota
                    _, stx = plsc.sort_key_val(mi, tix, descending=True)
                    wpos = pending + iota
                    plsc.store_scatter(locl, [wpos], sv)
                    plsc.store_scatter(tixl, [wpos], stx)
                    pending = pending + plsc.all_reduce_population_count(m)
                    full = pending >= 800

                    @pl.when(jnp.any(full))
                    def _():
                        fire()
                    return jnp.where(full, pending - 800, pending)
                pend = lax.fori_loop(0, half // 16, step, pend)

            @pl.when(jnp.any(pend > 0))
            def _():
                dumpv = jnp.full((16,), _DUMP, jnp.int32)
                zv = jnp.zeros((16,), jnp.int32)
                for j in range(50):
                    padpos = pend + j * 16 + iota
                    plsc.store_scatter(locl, [padpos], dumpv)
                    plsc.store_scatter(tixl, [padpos], zv)
                fire()
            plsc.subcore_barrier()

            o0 = pl.multiple_of(s * 5000, 8)
            pltpu.sync_copy(acc.at[pl.ds(o0, 5000)],
                            out_hbm.at[pl.ds(rng * _ER + o0, 5000)])
            plsc.subcore_barrier()

    return sk(y, rem)


def kernel(m_st, rbf, cbf, idx_swap, id3_kt, id3_st, id3_ragged_idx,
           W_mkt, W_rbf, W_down, W_bil, W_st, W_ts):
    del id3_ragged_idx
    E = m_st.shape[0]
    T = cbf.shape[0]
    eye8 = jnp.eye(8, dtype=jnp.float32)
    Wr_bd = jnp.kron(eye8, W_rbf).astype(jnp.bfloat16)
    Wd_bd = jnp.kron(eye8, W_down).astype(jnp.bfloat16)
    m16 = _stage_down(m_st, rbf.reshape(E // 8, 128), W_mkt,
                      Wr_bd, Wd_bd).reshape(E, 16)
    m_t = _gather_rows(m16, id3_kt, chunk=80, k=10)
    W2 = W_bil.reshape(16, 256)
    eye = jnp.eye(16, dtype=jnp.float32)
    Krep = jnp.repeat(eye, 16, axis=1)
    Ssum = jnp.tile(eye, (16, 1))
    W2_bd = jnp.kron(eye8, W2).astype(jnp.bfloat16)
    Kr_bd = jnp.kron(eye8, Krep).astype(jnp.bfloat16)
    Ss_bd = jnp.kron(eye8, Ssum).astype(jnp.bfloat16)
    ids2d = id3_st.reshape(T // _IDL, _IDL)
    y, rem = _stage_bilinear(cbf, m_t.reshape(T // 8, 128), ids2d,
                             W2_bd, Kr_bd, Ss_bd)
    x = _scatter_sum(y.reshape(T, 16), rem.reshape(4, T), E)
    x_sw = _gather_rows(x, idx_swap, chunk=80, k=5)
    return _stage_out(x.reshape(E // 8, 128), x_sw.reshape(E // 8, 128),
                      W_st, W_ts)


# speedup vs baseline: 1.0832x; 1.0632x over previous
import functools

import jax
import jax.numpy as jnp
from jax import lax
from jax.experimental import pallas as pl
from jax.experimental.pallas import tpu as pltpu
from jax.experimental.pallas import tpu_sc as plsc

INV_SQRT2 = 0.7071067811865476

NC = 2
NS = 16
NW = NC * NS


def _mesh():
    return plsc.VectorSubcoreMesh(core_axis_name="c", subcore_axis_name="s",
                                  num_cores=NC, num_subcores=NS)


def _pack8(g):
    n = g.shape[0] // 8
    a = g.reshape(n, 8, 16)
    return jnp.concatenate([a[:, k, :] for k in range(8)], axis=1)


def _unpack8(p):
    n = p.shape[0]
    parts = [p[:, 16 * k:16 * (k + 1)].reshape(n, 1, 16) for k in range(8)]
    return jnp.concatenate(parts, axis=1).reshape(n * 8, 16)


def _down_body(m_ref, rbfp_ref, wm_ref, wrbd_ref, wdbd_ref, out_ref):
    h = jax.nn.silu(jnp.dot(m_ref[...], wm_ref[...],
                            preferred_element_type=jnp.float32))
    Rr = out_ref.shape[0]
    h3 = h.reshape(Rr, 8, 128)
    hw = jnp.concatenate([h3[:, k, :] for k in range(8)], axis=1)
    rall = jnp.dot(rbfp_ref[...].astype(jnp.bfloat16), wrbd_ref[...],
                   preferred_element_type=jnp.float32)
    z = (hw * rall).astype(jnp.bfloat16)
    out_ref[...] = jax.nn.silu(jnp.dot(z, wdbd_ref[...],
                                       preferred_element_type=jnp.float32))


def _stage_down(m_st, rbf_p, W_mkt, Wr_bd, Wd_bd):
    E, DE = m_st.shape
    R = 3200
    return pl.pallas_call(
        _down_body,
        grid=(E // R,),
        in_specs=[
            pl.BlockSpec((R, DE), lambda i: (i, 0)),
            pl.BlockSpec((R // 8, 128), lambda i: (i, 0)),
            pl.BlockSpec(W_mkt.shape, lambda i: (0, 0)),
            pl.BlockSpec(Wr_bd.shape, lambda i: (0, 0)),
            pl.BlockSpec(Wd_bd.shape, lambda i: (0, 0)),
        ],
        out_specs=pl.BlockSpec((R // 8, 128), lambda i: (i, 0)),
        out_shape=jax.ShapeDtypeStruct((E // 8, 128), jnp.float32),
    )(m_st, rbf_p, W_mkt, Wr_bd, Wd_bd)


def _bil_body(cbfp_ref, mt_ref, ids_ref, w2bd_ref, krbd_ref, ssbd_ref,
              y_ref, rem_ref):
    P = jnp.dot(cbfp_ref[...].astype(jnp.bfloat16), krbd_ref[...],
                preferred_element_type=jnp.float32)
    M = jnp.dot(mt_ref[...].astype(jnp.bfloat16), w2bd_ref[...],
                preferred_element_type=jnp.float32)
    Z = (P * M).astype(jnp.bfloat16)
    y_ref[...] = jnp.dot(Z, ssbd_ref[...], preferred_element_type=jnp.float32)
    ids = ids_ref[...]
    for r in range(4):
        lo = r * _ER
        inr = (ids >= lo) & (ids < lo + _ER)
        rem_ref[r, :, :] = jnp.where(inr, ids - lo, _DUMP)


_IDL = 128


def _stage_bilinear(cbf_p, m_t_packed, ids2d, W2_bd, Kr_bd, Ss_bd):
    T = cbf_p.shape[0] * 8
    R = 5120
    return pl.pallas_call(
        _bil_body,
        grid=(T // R,),
        in_specs=[
            pl.BlockSpec((R // 8, 128), lambda i: (i, 0)),
            pl.BlockSpec((R // 8, 128), lambda i: (i, 0)),
            pl.BlockSpec((R // _IDL, _IDL), lambda i: (i, 0)),
            pl.BlockSpec(W2_bd.shape, lambda i: (0, 0)),
            pl.BlockSpec(Kr_bd.shape, lambda i: (0, 0)),
            pl.BlockSpec(Ss_bd.shape, lambda i: (0, 0)),
        ],
        out_specs=[
            pl.BlockSpec((R // 8, 128), lambda i: (i, 0)),
            pl.BlockSpec((4, R // _IDL, _IDL), lambda i: (0, i, 0)),
        ],
        out_shape=[
            jax.ShapeDtypeStruct((T // 8, 128), jnp.float32),
            jax.ShapeDtypeStruct((4, T // _IDL, _IDL), jnp.int32),
        ],
    )(cbf_p, m_t_packed, ids2d, W2_bd, Kr_bd, Ss_bd)


def _out_body(x_ref, xsw_ref, wst_ref, wts_ref, out_ref):
    xp = x_ref[...]
    xsp = xsw_ref[...]
    wst = wst_ref[...]
    wts = wts_ref[...]
    Rr = xp.shape[0]
    outs = []
    for k in range(8):
        a = jax.nn.silu(jnp.dot(xp[:, 16 * k:16 * (k + 1)], wst,
                                preferred_element_type=jnp.float32))
        b = jax.nn.silu(jnp.dot(xsp[:, 16 * k:16 * (k + 1)], wts,
                                preferred_element_type=jnp.float32))
        outs.append(((a + b) * INV_SQRT2).reshape(Rr, 1, wst.shape[1]))
    out_ref[...] = jnp.concatenate(outs, axis=1).reshape(out_ref.shape)


def _stage_out(x_packed, x_sw_packed, W_st, W_ts):
    E = x_packed.shape[0] * 8
    DE = W_st.shape[1]
    R = 3200
    return pl.pallas_call(
        _out_body,
        grid=(E // R,),
        in_specs=[
            pl.BlockSpec((R // 8, 128), lambda i: (i, 0)),
            pl.BlockSpec((R // 8, 128), lambda i: (i, 0)),
            pl.BlockSpec(W_st.shape, lambda i: (0, 0)),
            pl.BlockSpec(W_ts.shape, lambda i: (0, 0)),
        ],
        out_specs=pl.BlockSpec((R, DE), lambda i: (i, 0)),
        out_shape=jax.ShapeDtypeStruct((E, DE), jnp.float32),
    )(x_packed, x_sw_packed, W_st, W_ts)


def _gather_rows(table, idx, chunk, k):
    n = idx.shape[0]
    d = table.shape[1]
    per_w = n // NW
    n_groups = per_w // (chunk * k)

    @functools.partial(
        pl.kernel,
        out_type=jax.ShapeDtypeStruct((n, d), jnp.float32),
        mesh=_mesh(),
        compiler_params=pltpu.CompilerParams(use_tc_tiling_on_sc=False),
        scratch_types=[
            pltpu.VMEM((per_w,), jnp.int32),
            pltpu.VMEM((k * chunk, d), jnp.float32),
            pltpu.SemaphoreType.DMA,
        ],
    )
    def gk(table_hbm, idx_hbm, out_hbm, idx_v, buf, sem):
        wid = lax.axis_index("s") * NC + lax.axis_index("c")
        base = pl.multiple_of(wid * per_w, 8)
        pltpu.sync_copy(idx_hbm.at[pl.ds(base, per_w)], idx_v)

        def group(g, carry):
            descs = []
            for b in range(k):
                off = pl.multiple_of(g * (k * chunk) + b * chunk, 8)
                descs.append(pltpu.async_copy(
                    table_hbm.at[idx_v.at[pl.ds(off, chunk)]],
                    buf.at[pl.ds(b * chunk, chunk)], sem))
            for dsc in descs:
                dsc.wait()
            o = pl.multiple_of(base + g * (k * chunk), 8)
            pltpu.sync_copy(buf, out_hbm.at[pl.ds(o, k * chunk)])
            return carry

        lax.fori_loop(0, n_groups, group, 0)

    return gk(table, idx)


_ER = 80000
_ACC = 81920
_DUMP = 81000
_SCH = 800
_SJ = _SCH // 80


def _scatter_sum(y, rem, E):
    T, d = y.shape
    per_tile = T // NS
    half = per_tile // 2
    n_pass = E // (_ER * NC)

    @functools.partial(
        pl.kernel,
        out_type=jax.ShapeDtypeStruct((E, d), jnp.float32),
        mesh=_mesh(),
        compiler_params=pltpu.CompilerParams(use_tc_tiling_on_sc=False,
                                             needs_layout_passes=False),
        scratch_types=[
            pltpu.VMEM((half,), jnp.int32),
            pltpu.VMEM((1600,), jnp.int32),
            pltpu.VMEM((1600,), jnp.int32),
            pltpu.VMEM((800, d), jnp.float32),
            pltpu.VMEM((512, d), jnp.float32),
            pltpu.VMEM_SHARED((_ACC, d), jnp.float32),
            pltpu.SemaphoreType.DMA,
            pltpu.SemaphoreType.DMA,
        ],
    )
    def sk(y_hbm, rem_hbm, out_hbm, ib, locl, tixl, stg, zbuf, acc,
           lsem, ssem):
        c = lax.axis_index("c")
        s = lax.axis_index("s")
        t0 = s * per_tile
        iota = lax.iota(jnp.int32, 16)
        zero16 = jnp.zeros((d,), jnp.float32)

        def zrow(i, carry):
            zbuf[i, :] = zero16
            return carry
        lax.fori_loop(0, 512, zrow, 0)

        def fire():
            pltpu.async_copy(y_hbm.at[tixl.at[pl.ds(0, 800)]], stg,
                             lsem).wait()
            pltpu.async_copy(stg, acc.at[locl.at[pl.ds(0, 800)]], ssem,
                             add=True).wait()
            for j in range(50):
                o = 800 + j * 16
                locl[pl.ds(j * 16, 16)] = locl[pl.ds(o, 16)]
                tixl[pl.ds(j * 16, 16)] = tixl[pl.ds(o, 16)]

        for p in range(n_pass):
            rng = p * NC + c
            for zi in range(10):
                z0 = pl.multiple_of(s * 5120 + zi * 512, 8)
                pltpu.sync_copy(zbuf, acc.at[pl.ds(z0, 512)])
            plsc.subcore_barrier()

            pend = jnp.zeros((16,), jnp.int32)
            for hh in range(2):
                toff = pl.multiple_of(t0 + hh * half, 8)
                pltpu.sync_copy(rem_hbm.at[rng, pl.ds(toff, half)], ib)

                def step(j, pending):
                    v = ib[pl.ds(j * 16, 16)]
                    m = v < _ER
                    mi = m.astype(jnp.int32)
                    _, sv = plsc.sort_key_val(mi, v, descending=True)
                    tix = toff + j * 16 + iota
                    _, stx = plsc.sort_key_val(mi, tix, descending=True)
                    wpos = pending + iota
                    plsc.store_scatter(locl, [wpos], sv)
                    plsc.store_scatter(tixl, [wpos], stx)
                    pending = pending + plsc.all_reduce_population_count(m)
                    full = pending >= 800

                    @pl.when(jnp.any(full))
                    def _():
                        fire()
                    return jnp.where(full, pending - 800, pending)
                pend = lax.fori_loop(0, half // 16, step, pend)

            @pl.when(jnp.any(pend > 0))
            def _():
                dumpv = jnp.full((16,), _DUMP, jnp.int32)
                zv = jnp.zeros((16,), jnp.int32)
                for j in range(50):
                    padpos = pend + j * 16 + iota
                    plsc.store_scatter(locl, [padpos], dumpv)
                    plsc.store_scatter(tixl, [padpos], zv)
                fire()
            plsc.subcore_barrier()

            o0 = pl.multiple_of(s * 5000, 8)
            pltpu.sync_copy(acc.at[pl.ds(o0, 5000)],
                            out_hbm.at[pl.ds(rng * _ER + o0, 5000)])
            plsc.subcore_barrier()

    return sk(y, rem)


def kernel(m_st, rbf, cbf, idx_swap, id3_kt, id3_st, id3_ragged_idx,
           W_mkt, W_rbf, W_down, W_bil, W_st, W_ts):
    del id3_ragged_idx
    E = m_st.shape[0]
    T = cbf.shape[0]
    eye8 = jnp.eye(8, dtype=jnp.float32)
    Wr_bd = jnp.kron(eye8, W_rbf).astype(jnp.bfloat16)
    Wd_bd = jnp.kron(eye8, W_down).astype(jnp.bfloat16)
    m16 = _stage_down(m_st, rbf.reshape(E // 8, 128), W_mkt,
                      Wr_bd, Wd_bd).reshape(E, 16)
    m_t = _gather_rows(m16, id3_kt, chunk=80, k=10)
    W2 = W_bil.reshape(16, 256)
    eye = jnp.eye(16, dtype=jnp.float32)
    Krep = jnp.repeat(eye, 16, axis=1)
    Ssum = jnp.tile(eye, (16, 1))
    W2_bd = jnp.kron(eye8, W2).astype(jnp.bfloat16)
    Kr_bd = jnp.kron(eye8, Krep).astype(jnp.bfloat16)
    Ss_bd = jnp.kron(eye8, Ssum).astype(jnp.bfloat16)
    ids2d = id3_st.reshape(T // _IDL, _IDL)
    y, rem = _stage_bilinear(cbf.reshape(T // 8, 128),
                             m_t.reshape(T // 8, 128), ids2d,
                             W2_bd, Kr_bd, Ss_bd)
    x = _scatter_sum(y.reshape(T, 16), rem.reshape(4, T), E)
    x_sw = _gather_rows(x, idx_swap, chunk=80, k=5)
    return _stage_out(x.reshape(E // 8, 128), x_sw.reshape(E // 8, 128),
                      W_st, W_ts)
